# Initial kernel scaffold; baseline (speedup 1.0000x reference)
import jax
import jax.numpy as jnp
from jax.experimental import pallas as pl


def kernel(x, edge_index_r0, edge_index_r1, W1_0, b1_0, W1_1, b1_1, W2_0, b2_0, W2_1, b2_1):
    raise NotImplementedError("write your pallas kernel here")


# speedup vs baseline: 4.2403x; 4.2403x over previous
import dataclasses

import jax
import jax.numpy as jnp
from jax import lax
from jax.experimental import pallas as pl
from jax.experimental.pallas import tpu as pltpu
from jax.experimental.pallas import tpu_sc as plsc

N = 10000
D = 128
E = 160000
NSUB = 16
CH = 80
EPT = E // NSUB
NCHUNK = EPT // CH
NROWCH = N // CH
ROWK = -(-NROWCH // NSUB)
CSEG = 624


def _build_sc_agg():
    mesh = plsc.VectorSubcoreMesh(core_axis_name="c", subcore_axis_name="s")
    out_type = (jax.ShapeDtypeStruct((N, D), jnp.float32),
                jax.ShapeDtypeStruct((N, D), jnp.float32))
    scratch = (pltpu.VMEM((CH,), jnp.int32),
               pltpu.VMEM((CH,), jnp.int32),
               pltpu.VMEM((CH, D), jnp.float32),
               pltpu.SemaphoreType.DMA,
               pltpu.VMEM_SHARED((N, D), jnp.float32))

    def body(feat, src_all, dst_all, agg0, agg1, src_v, dst_v, rows_v, sem,
             acc_sh):
        c = lax.axis_index("c")
        s = lax.axis_index("s")

        def zrow(i, carry):
            for j in range(D // 16):
                rows_v[i, pl.ds(16 * j, 16)] = jnp.zeros((16,), jnp.float32)
            return carry
        lax.fori_loop(0, CH, zrow, 0)

        for ki in range(ROWK):
            k = s + NSUB * ki

            @pl.when(k < NROWCH)
            def _():
                pltpu.sync_copy(rows_v, acc_sh.at[pl.ds(k * CH, CH)])
        plsc.subcore_barrier()

        base0 = c * E + s * EPT

        def step(i, carry):
            b = base0 + i * CH
            pltpu.sync_copy(src_all.at[pl.ds(b, CH)], src_v)
            pltpu.sync_copy(dst_all.at[pl.ds(b, CH)], dst_v)
            pltpu.async_copy(feat.at[src_v], rows_v, sem).wait()
            pltpu.sync_copy(rows_v, acc_sh.at[dst_v], add=True)
            return carry
        lax.fori_loop(0, NCHUNK, step, 0)
        plsc.subcore_barrier()

        def writeout(agg_out):
            for ki in range(ROWK):
                k = s + NSUB * ki

                @pl.when(k < NROWCH)
                def _():
                    sl = pl.ds(k * CH, CH)
                    pltpu.sync_copy(acc_sh.at[sl], agg_out.at[sl])

        @pl.when(c == 0)
        def _():
            writeout(agg0)

        @pl.when(c == 1)
        def _():
            writeout(agg1)

    return pl.kernel(body, out_type=out_type, mesh=mesh,
                     scratch_types=scratch)


def _build_sc_count():
    mesh = plsc.VectorSubcoreMesh(core_axis_name="c", subcore_axis_name="s")
    out_type = (jax.ShapeDtypeStruct((N,), jnp.float32),
                jax.ShapeDtypeStruct((N,), jnp.float32))
    scratch = (pltpu.VMEM((CH,), jnp.int32),
               pltpu.VMEM((N,), jnp.float32),
               pltpu.VMEM((NSUB * (CSEG + 16),), jnp.float32),
               pltpu.VMEM((CSEG + 16,), jnp.float32),
               pltpu.VMEM_SHARED((NSUB * N,), jnp.float32))

    def body(dst_all, cnt0, cnt1, dst_v, hist_v, red_v, outc_v, hist_sh):
        c = lax.axis_index("c")
        s = lax.axis_index("s")

        def zhist(i, carry):
            hist_v[pl.ds(16 * i, 16)] = jnp.zeros((16,), jnp.float32)
            return carry
        lax.fori_loop(0, N // 16, zhist, 0)

        base0 = c * E + s * EPT

        def step(i, carry):
            b = base0 + i * CH
            pltpu.sync_copy(dst_all.at[pl.ds(b, CH)], dst_v)
            for j in range(CH // 16):
                d16 = dst_v[pl.ds(16 * j, 16)]
                rc, last = plsc.scan_count(d16)
                plsc.addupdate_scatter(hist_v, [d16],
                                       rc.astype(jnp.float32), mask=last)
            return carry
        lax.fori_loop(0, NCHUNK, step, 0)
        pltpu.sync_copy(hist_v, hist_sh.at[pl.ds(s * N, N)])
        plsc.subcore_barrier()

        def writeout(cnt_out):
            off = CSEG * s
            for t in range(NSUB):
                pltpu.sync_copy(hist_sh.at[pl.ds(t * N + off, CSEG + 16)],
                                red_v.at[pl.ds(t * (CSEG + 16), CSEG + 16)])

            def red(j, carry):
                v = red_v[pl.ds(16 * j, 16)]
                for t in range(1, NSUB):
                    v = v + red_v[pl.ds(t * (CSEG + 16) + 16 * j, 16)]
                outc_v[pl.ds(16 * j, 16)] = v
                return carry
            lax.fori_loop(0, (CSEG + 16) // 16, red, 0)
            pltpu.sync_copy(outc_v.at[pl.ds(0, CSEG)],
                            cnt_out.at[pl.ds(off, CSEG)])

            @pl.when(s == NSUB - 1)
            def _():
                pltpu.sync_copy(outc_v.at[pl.ds(CSEG, 16)],
                                cnt_out.at[pl.ds(off + CSEG, 16)])

        @pl.when(c == 0)
        def _():
            writeout(cnt0)

        @pl.when(c == 1)
        def _():
            writeout(cnt1)

    cp = pltpu.CompilerParams()
    if "needs_layout_passes" in pltpu.CompilerParams.__dataclass_fields__:
        cp = dataclasses.replace(cp, needs_layout_passes=False)
    return pl.kernel(body, out_type=out_type, mesh=mesh,
                     scratch_types=scratch, compiler_params=cp)


_sc_agg = _build_sc_agg()
_sc_count = _build_sc_count()

RB = 2000


def _build_tc_combine(first_layer):
    def body(a0, a1, c0, c1, w0, w1, b0, b1, o):
        cc0 = c0[...]
        cc1 = c1[...]
        x0 = a0[...] / jnp.maximum(cc0, 1.0)
        x1 = a1[...] / jnp.maximum(cc1, 1.0)
        t = jnp.dot(x0, w0[...], preferred_element_type=jnp.float32)
        t = t + jnp.dot(x1, w1[...], preferred_element_type=jnp.float32)
        t = t + jnp.where(cc0 > 0.0, 1.0, 0.0) * b0[...]
        t = t + jnp.where(cc1 > 0.0, 1.0, 0.0) * b1[...]
        if first_layer:
            t = jnp.where(t >= 0, t, 0.01 * t)
        o[...] = t

    return pl.pallas_call(
        body,
        grid=(N // RB,),
        in_specs=[pl.BlockSpec((RB, D), lambda i: (i, 0)),
                  pl.BlockSpec((RB, D), lambda i: (i, 0)),
                  pl.BlockSpec((RB, 1), lambda i: (i, 0)),
                  pl.BlockSpec((RB, 1), lambda i: (i, 0)),
                  pl.BlockSpec((D, D), lambda i: (0, 0)),
                  pl.BlockSpec((D, D), lambda i: (0, 0)),
                  pl.BlockSpec((1, D), lambda i: (0, 0)),
                  pl.BlockSpec((1, D), lambda i: (0, 0))],
        out_specs=pl.BlockSpec((RB, D), lambda i: (i, 0)),
        out_shape=jax.ShapeDtypeStruct((N, D), jnp.float32),
    )


_tc_combine1 = _build_tc_combine(True)
_tc_combine2 = _build_tc_combine(False)


def kernel(x, edge_index_r0, edge_index_r1, W1_0, b1_0, W1_1, b1_1,
           W2_0, b2_0, W2_1, b2_1):
    ei0 = edge_index_r0.astype(jnp.int32)
    ei1 = edge_index_r1.astype(jnp.int32)
    src_all = jnp.concatenate([ei0[0], ei1[0]])
    dst_all = jnp.concatenate([ei0[1], ei1[1]])
    agg0, agg1 = _sc_agg(x, src_all, dst_all)
    cnt0, cnt1 = _sc_count(dst_all)
    cnt0 = cnt0.reshape(N, 1)
    cnt1 = cnt1.reshape(N, 1)
    h = _tc_combine1(agg0, agg1, cnt0, cnt1, W1_0, W1_1,
                     b1_0.reshape(1, D), b1_1.reshape(1, D))
    agg0b, agg1b = _sc_agg(h, src_all, dst_all)
    out = _tc_combine2(agg0b, agg1b, cnt0, cnt1, W2_0, W2_1,
                       b2_0.reshape(1, D), b2_1.reshape(1, D))
    return out


# speedup vs baseline: 7.4757x; 1.7630x over previous
import dataclasses

import jax
import jax.numpy as jnp
from jax import lax
from jax.experimental import pallas as pl
from jax.experimental.pallas import tpu as pltpu
from jax.experimental.pallas import tpu_sc as plsc

N = 10000
D = 128
E = 160000
NSUB = 16
CH = 80
EPT = E // NSUB
NCHUNK = EPT // CH
NPAIR = (NCHUNK - 1) // 2
NROWCH = N // CH
ROWK = -(-NROWCH // NSUB)
CSEG = 624


def _build_sc_agg():
    mesh = plsc.VectorSubcoreMesh(core_axis_name="c", subcore_axis_name="s")
    out_type = (jax.ShapeDtypeStruct((N, D), jnp.float32),
                jax.ShapeDtypeStruct((N, D), jnp.float32))
    scratch = (pltpu.VMEM((EPT,), jnp.int32),
               pltpu.VMEM((EPT,), jnp.int32),
               pltpu.VMEM((CH,), jnp.int32),
               pltpu.VMEM((CH,), jnp.int32),
               pltpu.VMEM((CH, D), jnp.float32),
               pltpu.VMEM((CH, D), jnp.float32),
               pltpu.SemaphoreType.DMA,
               pltpu.SemaphoreType.DMA,
               pltpu.SemaphoreType.DMA,
               pltpu.SemaphoreType.DMA,
               pltpu.VMEM_SHARED((N, D), jnp.float32))

    def body(feat, src_all, dst_all, zrows, agg0, agg1, src_b, dst_b,
             dv0, dv1, rows0, rows1, gs0, gs1, ss0, ss1, acc_sh):
        with_counts = False
        c = lax.axis_index("c")
        s = lax.axis_index("s")

        base0 = c * E + s * EPT
        pltpu.sync_copy(src_all.at[pl.ds(base0, EPT)], src_b)
        pltpu.sync_copy(dst_all.at[pl.ds(base0, EPT)], dst_b)

        for ki in range(ROWK):
            k = s + NSUB * ki

            @pl.when(k < NROWCH)
            def _():
                pltpu.async_copy(zrows, acc_sh.at[pl.ds(k * CH, CH)],
                                 gs0).wait()
        if with_counts:
            def zhist(i, carry):
                hist_v[pl.ds(16 * i, 16)] = jnp.zeros((16,), jnp.float32)
                return carry
            lax.fori_loop(0, N // 16, zhist, 0)
        plsc.subcore_barrier()

        def count_chunk(i):
            if not with_counts:
                return
            for j in range(CH // 16):
                d16 = dst_b[pl.ds(i * CH + 16 * j, 16)]
                rc, last = plsc.scan_count(d16)
                plsc.addupdate_scatter(hist_v, [d16],
                                       rc.astype(jnp.float32), mask=last)

        def idx_of(i):
            return src_b.at[pl.ds(i * CH, CH)]

        def start_gather(i, rows, gsem):
            pltpu.async_copy(feat.at[idx_of(i)], rows, gsem)

        def wait_gather(rows, gsem):
            pltpu.make_async_copy(feat.at[idx_of(0)], rows, gsem).wait()

        def start_scatter(i, dv, rows, ssem):
            for j in range(CH // 16):
                dv[pl.ds(16 * j, 16)] = dst_b[pl.ds(i * CH + 16 * j, 16)]
            pltpu.async_copy(rows, acc_sh.at[dv], ssem, add=True)

        def wait_scatter(dv, rows, ssem):
            pltpu.make_async_copy(rows, acc_sh.at[dv], ssem).wait()

        start_gather(0, rows0, gs0)

        def pair(j, carry):
            a = 2 * j
            wait_gather(rows0, gs0)
            start_scatter(a, dv0, rows0, ss0)
            start_gather(a + 1, rows1, gs1)
            count_chunk(a)
            wait_gather(rows1, gs1)
            start_scatter(a + 1, dv1, rows1, ss1)
            wait_scatter(dv0, rows0, ss0)
            start_gather(a + 2, rows0, gs0)
            count_chunk(a + 1)
            wait_scatter(dv1, rows1, ss1)
            return carry
        lax.fori_loop(0, NPAIR, pair, 0)
        wait_gather(rows0, gs0)
        start_scatter(NCHUNK - 1, dv0, rows0, ss0)
        count_chunk(NCHUNK - 1)
        wait_scatter(dv0, rows0, ss0)
        if with_counts:
            pltpu.sync_copy(hist_v, hist_sh.at[pl.ds(s * N, N)])
        plsc.subcore_barrier()

        def writeout(agg_out):
            for ki in range(ROWK):
                k = s + NSUB * ki

                @pl.when(k < NROWCH)
                def _():
                    sl = pl.ds(k * CH, CH)
                    pltpu.sync_copy(acc_sh.at[sl], agg_out.at[sl])

        @pl.when(c == 0)
        def _():
            writeout(agg0)

        @pl.when(c == 1)
        def _():
            writeout(agg1)

    return pl.kernel(body, out_type=out_type, mesh=mesh,
                     scratch_types=scratch)


def _build_sc_count():
    mesh = plsc.VectorSubcoreMesh(core_axis_name="c", subcore_axis_name="s")
    out_type = (jax.ShapeDtypeStruct((N,), jnp.float32),
                jax.ShapeDtypeStruct((N,), jnp.float32))
    scratch = (pltpu.VMEM((EPT,), jnp.int32),
               pltpu.VMEM((N,), jnp.float32),
               pltpu.VMEM((NSUB * (CSEG + 16),), jnp.float32),
               pltpu.VMEM((CSEG + 16,), jnp.float32),
               pltpu.VMEM_SHARED((NSUB * N,), jnp.float32))

    def body(dst_all, cnt0, cnt1, dst_b, hist_v, red_v, outc_v, hist_sh):
        c = lax.axis_index("c")
        s = lax.axis_index("s")
        pltpu.sync_copy(dst_all.at[pl.ds(c * E + s * EPT, EPT)], dst_b)

        def zhist(i, carry):
            hist_v[pl.ds(16 * i, 16)] = jnp.zeros((16,), jnp.float32)
            return carry
        lax.fori_loop(0, N // 16, zhist, 0)

        def step(i, carry):
            d16 = dst_b[pl.ds(16 * i, 16)]
            rc, last = plsc.scan_count(d16)
            plsc.addupdate_scatter(hist_v, [d16],
                                   rc.astype(jnp.float32), mask=last)
            return carry
        lax.fori_loop(0, EPT // 16, step, 0)
        pltpu.sync_copy(hist_v, hist_sh.at[pl.ds(s * N, N)])
        plsc.subcore_barrier()

        def writeout(cnt_out):
            off = CSEG * s
            for t in range(NSUB):
                pltpu.sync_copy(hist_sh.at[pl.ds(t * N + off, CSEG + 16)],
                                red_v.at[pl.ds(t * (CSEG + 16), CSEG + 16)])

            def red(j, carry):
                v = red_v[pl.ds(16 * j, 16)]
                for t in range(1, NSUB):
                    v = v + red_v[pl.ds(t * (CSEG + 16) + 16 * j, 16)]
                outc_v[pl.ds(16 * j, 16)] = v
                return carry
            lax.fori_loop(0, (CSEG + 16) // 16, red, 0)
            pltpu.sync_copy(outc_v.at[pl.ds(0, CSEG)],
                            cnt_out.at[pl.ds(off, CSEG)])

            @pl.when(s == NSUB - 1)
            def _():
                pltpu.sync_copy(outc_v.at[pl.ds(CSEG, 16)],
                                cnt_out.at[pl.ds(off + CSEG, 16)])

        @pl.when(c == 0)
        def _():
            writeout(cnt0)

        @pl.when(c == 1)
        def _():
            writeout(cnt1)

    cp = pltpu.CompilerParams()
    if "needs_layout_passes" in pltpu.CompilerParams.__dataclass_fields__:
        cp = dataclasses.replace(cp, needs_layout_passes=False)
    return pl.kernel(body, out_type=out_type, mesh=mesh,
                     scratch_types=scratch, compiler_params=cp)


_sc_agg = _build_sc_agg()
_sc_count = _build_sc_count()

RB = 2000


def _build_tc_combine(first_layer):
    def body(a0, a1, c0, c1, w0, w1, b0, b1, o):
        cc0 = c0[...]
        cc1 = c1[...]
        x0 = a0[...] / jnp.maximum(cc0, 1.0)
        x1 = a1[...] / jnp.maximum(cc1, 1.0)
        t = jnp.dot(x0, w0[...], preferred_element_type=jnp.float32)
        t = t + jnp.dot(x1, w1[...], preferred_element_type=jnp.float32)
        t = t + jnp.where(cc0 > 0.0, 1.0, 0.0) * b0[...]
        t = t + jnp.where(cc1 > 0.0, 1.0, 0.0) * b1[...]
        if first_layer:
            t = jnp.where(t >= 0, t, 0.01 * t)
        o[...] = t

    return pl.pallas_call(
        body,
        grid=(N // RB,),
        in_specs=[pl.BlockSpec((RB, D), lambda i: (i, 0)),
                  pl.BlockSpec((RB, D), lambda i: (i, 0)),
                  pl.BlockSpec((RB, 1), lambda i: (i, 0)),
                  pl.BlockSpec((RB, 1), lambda i: (i, 0)),
                  pl.BlockSpec((D, D), lambda i: (0, 0)),
                  pl.BlockSpec((D, D), lambda i: (0, 0)),
                  pl.BlockSpec((1, D), lambda i: (0, 0)),
                  pl.BlockSpec((1, D), lambda i: (0, 0))],
        out_specs=pl.BlockSpec((RB, D), lambda i: (i, 0)),
        out_shape=jax.ShapeDtypeStruct((N, D), jnp.float32),
    )


_tc_combine1 = _build_tc_combine(True)
_tc_combine2 = _build_tc_combine(False)


def kernel(x, edge_index_r0, edge_index_r1, W1_0, b1_0, W1_1, b1_1,
           W2_0, b2_0, W2_1, b2_1):
    ei0 = edge_index_r0.astype(jnp.int32)
    ei1 = edge_index_r1.astype(jnp.int32)
    src_all = jnp.concatenate([ei0[0], ei1[0]])
    dst_all = jnp.concatenate([ei0[1], ei1[1]])
    zrows = jnp.zeros((CH, D), jnp.float32)
    agg0, agg1 = _sc_agg(x, src_all, dst_all, zrows)
    cnt0, cnt1 = _sc_count(dst_all)
    cnt0 = cnt0.reshape(N, 1)
    cnt1 = cnt1.reshape(N, 1)
    h = _tc_combine1(agg0, agg1, cnt0, cnt1, W1_0, W1_1,
                     b1_0.reshape(1, D), b1_1.reshape(1, D))
    agg0b, agg1b = _sc_agg(h, src_all, dst_all, zrows)
    out = _tc_combine2(agg0b, agg1b, cnt0, cnt1, W2_0, W2_1,
                       b2_0.reshape(1, D), b2_1.reshape(1, D))
    return out


# speedup vs baseline: 8.2780x; 1.1073x over previous
import dataclasses

import jax
import jax.numpy as jnp
from jax import lax
from jax.experimental import pallas as pl
from jax.experimental.pallas import tpu as pltpu
from jax.experimental.pallas import tpu_sc as plsc

N = 10000
D = 128
E = 160000
NSUB = 16
CH = 80
EPT = E // NSUB
NCHUNK = EPT // CH
NPAIR = (NCHUNK - 1) // 2
NROWCH = N // CH
ROWK = -(-NROWCH // NSUB)
CSEG = 624


def _build_sc_agg():
    mesh = plsc.VectorSubcoreMesh(core_axis_name="c", subcore_axis_name="s")
    out_type = (jax.ShapeDtypeStruct((N, D), jnp.float32),
                jax.ShapeDtypeStruct((N, D), jnp.float32))
    scratch = ([pltpu.VMEM((CH,), jnp.int32) for _ in range(4)],
               [pltpu.VMEM((CH,), jnp.int32) for _ in range(4)],
               [pltpu.VMEM((CH, D), jnp.float32) for _ in range(4)],
               pltpu.SemaphoreType.DMA((4,)),
               pltpu.SemaphoreType.DMA((4,)),
               pltpu.SemaphoreType.DMA((4,)),
               pltpu.VMEM_SHARED((N, D), jnp.float32))

    def body(feat, src_all, dst_all, agg0, agg1, sv, dv, rows,
             isem, gs, ss, acc_sh):
        c = lax.axis_index("c")
        s = lax.axis_index("s")
        base0 = c * E + s * EPT

        def start_idx(i, b):
            pltpu.async_copy(src_all.at[pl.ds(base0 + i * CH, CH)], sv[b],
                             isem.at[b])
            pltpu.async_copy(dst_all.at[pl.ds(base0 + i * CH, CH)], dv[b],
                             isem.at[b])

        def wait_idx(b):
            pltpu.make_async_copy(src_all.at[pl.ds(0, CH)], sv[b],
                                  isem.at[b]).wait()
            pltpu.make_async_copy(dst_all.at[pl.ds(0, CH)], dv[b],
                                  isem.at[b]).wait()

        def start_gather(i, b):
            pltpu.async_copy(feat.at[sv[b]], rows[b], gs.at[b])

        def wait_gather(b):
            pltpu.make_async_copy(feat.at[sv[b]], rows[b], gs.at[b]).wait()

        def start_scatter(i, b):
            pltpu.async_copy(rows[b], acc_sh.at[dv[b]], ss.at[b], add=True)

        def wait_scatter(b):
            pltpu.make_async_copy(rows[b], acc_sh.at[dv[b]], ss.at[b]).wait()

        for i in range(4):
            start_idx(i, i)

        def zrow(i, carry):
            for j in range(D // 16):
                rows[0][i, pl.ds(16 * j, 16)] = jnp.zeros((16,), jnp.float32)
            return carry
        lax.fori_loop(0, CH, zrow, 0)
        for ki in range(ROWK):
            k = s + NSUB * ki

            @pl.when(k < NROWCH)
            def _():
                pltpu.async_copy(rows[0], acc_sh.at[pl.ds(k * CH, CH)],
                                 ss.at[0])
        for ki in range(ROWK):
            k = s + NSUB * ki

            @pl.when(k < NROWCH)
            def _():
                pltpu.make_async_copy(
                    rows[0], acc_sh.at[pl.ds(k * CH, CH)], ss.at[0]).wait()
        plsc.subcore_barrier()

        wait_idx(0)
        start_gather(0, 0)
        wait_idx(1)
        start_gather(1, 1)
        wait_gather(0)
        start_scatter(0, 0)
        wait_idx(2)
        start_gather(2, 2)
        wait_gather(1)
        start_scatter(1, 1)

        def unit(i, b):
            wait_gather(b)
            start_scatter(i, b)
            wait_scatter((b - 2) % 4)
            start_idx(i + 2, (b + 2) % 4)
            wait_idx((b + 1) % 4)
            start_gather(i + 1, (b + 1) % 4)

        def trip(j, carry):
            i0 = 4 * j + 2
            for k in range(4):
                unit(i0 + k, (2 + k) % 4)
            return carry
        NTRIP = (NCHUNK - 5) // 4
        lax.fori_loop(0, NTRIP, trip, 0)
        i0 = 4 * NTRIP + 2
        b0 = i0 % 4
        unit(i0, b0)
        wait_gather((b0 + 1) % 4)
        start_scatter(i0 + 1, (b0 + 1) % 4)
        wait_scatter((b0 - 1) % 4)
        wait_idx((b0 + 2) % 4)
        start_gather(i0 + 2, (b0 + 2) % 4)
        wait_gather((b0 + 2) % 4)
        start_scatter(i0 + 2, (b0 + 2) % 4)
        wait_scatter(b0)
        wait_scatter((b0 + 1) % 4)
        wait_scatter((b0 + 2) % 4)
        plsc.subcore_barrier()

        def writeout(agg_out):
            for ki in range(ROWK):
                k = s + NSUB * ki

                @pl.when(k < NROWCH)
                def _():
                    sl = pl.ds(k * CH, CH)
                    pltpu.sync_copy(acc_sh.at[sl], agg_out.at[sl])

        @pl.when(c == 0)
        def _():
            writeout(agg0)

        @pl.when(c == 1)
        def _():
            writeout(agg1)

    return pl.kernel(body, out_type=out_type, mesh=mesh,
                     scratch_types=scratch)


def _build_sc_count():
    mesh = plsc.VectorSubcoreMesh(core_axis_name="c", subcore_axis_name="s")
    out_type = (jax.ShapeDtypeStruct((N,), jnp.float32),
                jax.ShapeDtypeStruct((N,), jnp.float32))
    scratch = (pltpu.VMEM((EPT,), jnp.int32),
               pltpu.VMEM((N,), jnp.float32),
               pltpu.VMEM((NSUB * (CSEG + 16),), jnp.float32),
               pltpu.VMEM((CSEG + 16,), jnp.float32),
               pltpu.VMEM_SHARED((NSUB * N,), jnp.float32))

    def body(dst_all, cnt0, cnt1, dst_b, hist_v, red_v, outc_v, hist_sh):
        c = lax.axis_index("c")
        s = lax.axis_index("s")
        pltpu.sync_copy(dst_all.at[pl.ds(c * E + s * EPT, EPT)], dst_b)

        def zhist(i, carry):
            hist_v[pl.ds(16 * i, 16)] = jnp.zeros((16,), jnp.float32)
            return carry
        lax.fori_loop(0, N // 16, zhist, 0)

        def step(i, carry):
            d16 = dst_b[pl.ds(16 * i, 16)]
            rc, last = plsc.scan_count(d16)
            plsc.addupdate_scatter(hist_v, [d16],
                                   rc.astype(jnp.float32), mask=last)
            return carry
        lax.fori_loop(0, EPT // 16, step, 0)
        pltpu.sync_copy(hist_v, hist_sh.at[pl.ds(s * N, N)])
        plsc.subcore_barrier()

        def writeout(cnt_out):
            off = CSEG * s
            for t in range(NSUB):
                pltpu.sync_copy(hist_sh.at[pl.ds(t * N + off, CSEG + 16)],
                                red_v.at[pl.ds(t * (CSEG + 16), CSEG + 16)])

            def red(j, carry):
                v = red_v[pl.ds(16 * j, 16)]
                for t in range(1, NSUB):
                    v = v + red_v[pl.ds(t * (CSEG + 16) + 16 * j, 16)]
                outc_v[pl.ds(16 * j, 16)] = v
                return carry
            lax.fori_loop(0, (CSEG + 16) // 16, red, 0)
            pltpu.sync_copy(outc_v.at[pl.ds(0, CSEG)],
                            cnt_out.at[pl.ds(off, CSEG)])

            @pl.when(s == NSUB - 1)
            def _():
                pltpu.sync_copy(outc_v.at[pl.ds(CSEG, 16)],
                                cnt_out.at[pl.ds(off + CSEG, 16)])

        @pl.when(c == 0)
        def _():
            writeout(cnt0)

        @pl.when(c == 1)
        def _():
            writeout(cnt1)

    cp = pltpu.CompilerParams()
    if "needs_layout_passes" in pltpu.CompilerParams.__dataclass_fields__:
        cp = dataclasses.replace(cp, needs_layout_passes=False)
    return pl.kernel(body, out_type=out_type, mesh=mesh,
                     scratch_types=scratch, compiler_params=cp)


_sc_agg = _build_sc_agg()
_sc_count = _build_sc_count()

RB = 2000


def _build_tc_combine(first_layer):
    def body(a0, a1, c0, c1, w0, w1, b0, b1, o):
        cc0 = c0[...]
        cc1 = c1[...]
        x0 = a0[...] / jnp.maximum(cc0, 1.0)
        x1 = a1[...] / jnp.maximum(cc1, 1.0)
        t = jnp.dot(x0, w0[...], preferred_element_type=jnp.float32)
        t = t + jnp.dot(x1, w1[...], preferred_element_type=jnp.float32)
        t = t + jnp.where(cc0 > 0.0, 1.0, 0.0) * b0[...]
        t = t + jnp.where(cc1 > 0.0, 1.0, 0.0) * b1[...]
        if first_layer:
            t = jnp.where(t >= 0, t, 0.01 * t)
        o[...] = t

    return pl.pallas_call(
        body,
        grid=(N // RB,),
        in_specs=[pl.BlockSpec((RB, D), lambda i: (i, 0)),
                  pl.BlockSpec((RB, D), lambda i: (i, 0)),
                  pl.BlockSpec((RB, 1), lambda i: (i, 0)),
                  pl.BlockSpec((RB, 1), lambda i: (i, 0)),
                  pl.BlockSpec((D, D), lambda i: (0, 0)),
                  pl.BlockSpec((D, D), lambda i: (0, 0)),
                  pl.BlockSpec((1, D), lambda i: (0, 0)),
                  pl.BlockSpec((1, D), lambda i: (0, 0))],
        out_specs=pl.BlockSpec((RB, D), lambda i: (i, 0)),
        out_shape=jax.ShapeDtypeStruct((N, D), jnp.float32),
    )


_tc_combine1 = _build_tc_combine(True)
_tc_combine2 = _build_tc_combine(False)


def kernel(x, edge_index_r0, edge_index_r1, W1_0, b1_0, W1_1, b1_1,
           W2_0, b2_0, W2_1, b2_1):
    ei0 = edge_index_r0.astype(jnp.int32)
    ei1 = edge_index_r1.astype(jnp.int32)
    src_all = jnp.concatenate([ei0[0], ei1[0]])
    dst_all = jnp.concatenate([ei0[1], ei1[1]])
    agg0, agg1 = _sc_agg(x, src_all, dst_all)
    cnt0, cnt1 = _sc_count(dst_all)
    cnt0 = cnt0.reshape(N, 1)
    cnt1 = cnt1.reshape(N, 1)
    h = _tc_combine1(agg0, agg1, cnt0, cnt1, W1_0, W1_1,
                     b1_0.reshape(1, D), b1_1.reshape(1, D))
    agg0b, agg1b = _sc_agg(h, src_all, dst_all)
    out = _tc_combine2(agg0b, agg1b, cnt0, cnt1, W2_0, W2_1,
                       b2_0.reshape(1, D), b2_1.reshape(1, D))
    return out


# speedup vs baseline: 8.3624x; 1.0102x over previous
import dataclasses

import jax
import jax.numpy as jnp
from jax import lax
from jax.experimental import pallas as pl
from jax.experimental.pallas import tpu as pltpu
from jax.experimental.pallas import tpu_sc as plsc

N = 10000
D = 128
E = 160000
NSUB = 16
CH = 80
EPT = E // NSUB
NCHUNK = EPT // CH
NPAIR = (NCHUNK - 1) // 2
NROWCH = N // CH
ROWK = -(-NROWCH // NSUB)
CSEG = 624


def _build_sc_agg():
    mesh = plsc.VectorSubcoreMesh(core_axis_name="c", subcore_axis_name="s")
    out_type = (jax.ShapeDtypeStruct((N, D), jnp.float32),
                jax.ShapeDtypeStruct((N, D), jnp.float32))
    scratch = ([pltpu.VMEM((CH,), jnp.int32) for _ in range(2)],
               [pltpu.VMEM((CH,), jnp.int32) for _ in range(4)],
               [pltpu.VMEM((4 * CH,), jnp.int32) for _ in range(2)],
               [pltpu.VMEM((4 * CH,), jnp.int32) for _ in range(2)],
               [pltpu.VMEM((CH, D), jnp.float32) for _ in range(4)],
               pltpu.SemaphoreType.DMA((4,)),
               pltpu.SemaphoreType.DMA((2,)),
               pltpu.SemaphoreType.DMA((4,)),
               pltpu.SemaphoreType.DMA((4,)),
               pltpu.VMEM_SHARED((N, D), jnp.float32))

    def body(feat, src_all, dst_all, agg0, agg1, sv, dv, sb, db, rows,
             isem, bsem, gs, ss, acc_sh):
        c = lax.axis_index("c")
        s = lax.axis_index("s")
        base0 = c * E + s * EPT

        def start_idx(i, b):
            pltpu.async_copy(src_all.at[pl.ds(base0 + i * CH, CH)], sv[b],
                             isem.at[b])
            pltpu.async_copy(dst_all.at[pl.ds(base0 + i * CH, CH)], dv[b],
                             isem.at[b])

        def wait_idx(b):
            pltpu.make_async_copy(src_all.at[pl.ds(0, CH)], sv[b],
                                  isem.at[b]).wait()
            pltpu.make_async_copy(dst_all.at[pl.ds(0, CH)], dv[b],
                                  isem.at[b]).wait()

        def start_group(g, p):
            o = base0 + (4 * g + 2) * CH
            pltpu.async_copy(src_all.at[pl.ds(o, 4 * CH)], sb[p], bsem.at[p])
            pltpu.async_copy(dst_all.at[pl.ds(o, 4 * CH)], db[p], bsem.at[p])

        def wait_group(p):
            pltpu.make_async_copy(src_all.at[pl.ds(0, 4 * CH)], sb[p],
                                  bsem.at[p]).wait()
            pltpu.make_async_copy(dst_all.at[pl.ds(0, 4 * CH)], db[p],
                                  bsem.at[p]).wait()

        def start_gather_sv(b):
            pltpu.async_copy(feat.at[sv[b]], rows[b], gs.at[b])

        def start_gather(b, p, off):
            pltpu.async_copy(feat.at[sb[p].at[pl.ds(off * CH, CH)]], rows[b],
                             gs.at[b])

        def wait_gather(b):
            pltpu.make_async_copy(feat.at[sv[0]], rows[b], gs.at[b]).wait()

        def start_scatter(b):
            pltpu.async_copy(rows[b], acc_sh.at[dv[b]], ss.at[b], add=True)

        def wait_scatter(b):
            pltpu.make_async_copy(rows[b], acc_sh.at[dv[b]], ss.at[b]).wait()

        start_idx(0, 0)
        start_idx(1, 1)
        pltpu.async_copy(dst_all.at[pl.ds(base0, CH)], dv[3], isem.at[3])
        start_group(0, 0)
        start_group(1, 1)

        def zrow(i, carry):
            for j in range(D // 16):
                rows[0][i, pl.ds(16 * j, 16)] = jnp.zeros((16,), jnp.float32)
                rows[3][i, pl.ds(16 * j, 16)] = jnp.zeros((16,), jnp.float32)
            return carry
        lax.fori_loop(0, CH, zrow, 0)
        for ki in range(ROWK):
            k = s + NSUB * ki

            @pl.when(k < NROWCH)
            def _():
                pltpu.async_copy(rows[0], acc_sh.at[pl.ds(k * CH, CH)],
                                 ss.at[0])
        for ki in range(ROWK):
            k = s + NSUB * ki

            @pl.when(k < NROWCH)
            def _():
                pltpu.make_async_copy(
                    rows[0], acc_sh.at[pl.ds(k * CH, CH)], ss.at[0]).wait()
        plsc.subcore_barrier()

        pltpu.make_async_copy(dst_all.at[pl.ds(0, CH)], dv[3],
                              isem.at[3]).wait()
        pltpu.async_copy(rows[3], acc_sh.at[dv[3]], ss.at[3], add=True)
        wait_idx(0)
        start_gather_sv(0)
        wait_idx(1)
        start_gather_sv(1)
        wait_gather(0)
        start_scatter(0)
        wait_group(0)
        start_gather(2, 0, 0)
        wait_gather(1)
        start_scatter(1)

        def copy_dst(b, p, off):
            for j in range(CH // 16):
                dv[b][pl.ds(16 * j, 16)] = db[p][pl.ds(off * CH + 16 * j, 16)]

        def unit(b, p, off, nb, np_, noff):
            wait_gather(b)
            copy_dst(b, p, off)
            start_scatter(b)
            wait_scatter((b + 1) % 4)
            start_gather(nb, np_, noff)

        def iteration(j, carry):
            unit(2, 0, 0, 3, 0, 1)
            unit(3, 0, 1, 0, 0, 2)
            unit(0, 0, 2, 1, 0, 3)
            wait_gather(1)
            copy_dst(1, 0, 3)
            start_scatter(1)
            wait_scatter(2)
            wait_group(1)
            start_gather(2, 1, 0)
            start_group(2 * j + 2, 0)
            unit(2, 1, 0, 3, 1, 1)
            unit(3, 1, 1, 0, 1, 2)
            unit(0, 1, 2, 1, 1, 3)
            wait_gather(1)
            copy_dst(1, 1, 3)
            start_scatter(1)
            wait_scatter(2)
            wait_group(0)
            start_gather(2, 0, 0)
            start_group(2 * j + 3, 1)
            return carry
        NIT = (NCHUNK - 5) // 8
        lax.fori_loop(0, NIT, iteration, 0)
        unit(2, 0, 0, 3, 0, 1)
        unit(3, 0, 1, 0, 0, 2)
        wait_gather(0)
        copy_dst(0, 0, 2)
        start_scatter(0)
        wait_scatter(1)
        wait_scatter(2)
        wait_scatter(3)
        wait_scatter(0)
        wait_group(1)
        plsc.subcore_barrier()

        def writeout(agg_out):
            for ki in range(ROWK):
                k = s + NSUB * ki

                @pl.when(k < NROWCH)
                def _():
                    sl = pl.ds(k * CH, CH)
                    pltpu.sync_copy(acc_sh.at[sl], agg_out.at[sl])

        @pl.when(c == 0)
        def _():
            writeout(agg0)

        @pl.when(c == 1)
        def _():
            writeout(agg1)

    return pl.kernel(body, out_type=out_type, mesh=mesh,
                     scratch_types=scratch)


def _build_sc_count():
    mesh = plsc.VectorSubcoreMesh(core_axis_name="c", subcore_axis_name="s")
    out_type = (jax.ShapeDtypeStruct((N,), jnp.float32),
                jax.ShapeDtypeStruct((N,), jnp.float32))
    scratch = (pltpu.VMEM((EPT,), jnp.int32),
               pltpu.VMEM((N,), jnp.float32),
               pltpu.VMEM((NSUB * (CSEG + 16),), jnp.float32),
               pltpu.VMEM((CSEG + 16,), jnp.float32),
               pltpu.VMEM_SHARED((NSUB * N,), jnp.float32))

    def body(dst_all, cnt0, cnt1, dst_b, hist_v, red_v, outc_v, hist_sh):
        c = lax.axis_index("c")
        s = lax.axis_index("s")
        pltpu.sync_copy(dst_all.at[pl.ds(c * E + s * EPT, EPT)], dst_b)

        def zhist(i, carry):
            hist_v[pl.ds(16 * i, 16)] = jnp.zeros((16,), jnp.float32)
            return carry
        lax.fori_loop(0, N // 16, zhist, 0)

        def step(i, carry):
            d16 = dst_b[pl.ds(16 * i, 16)]
            rc, last = plsc.scan_count(d16)
            plsc.addupdate_scatter(hist_v, [d16],
                                   rc.astype(jnp.float32), mask=last)
            return carry
        lax.fori_loop(0, EPT // 16, step, 0)
        pltpu.sync_copy(hist_v, hist_sh.at[pl.ds(s * N, N)])
        plsc.subcore_barrier()

        def writeout(cnt_out):
            off = CSEG * s
            for t in range(NSUB):
                pltpu.sync_copy(hist_sh.at[pl.ds(t * N + off, CSEG + 16)],
                                red_v.at[pl.ds(t * (CSEG + 16), CSEG + 16)])

            def red(j, carry):
                v = red_v[pl.ds(16 * j, 16)]
                for t in range(1, NSUB):
                    v = v + red_v[pl.ds(t * (CSEG + 16) + 16 * j, 16)]
                outc_v[pl.ds(16 * j, 16)] = v
                return carry
            lax.fori_loop(0, (CSEG + 16) // 16, red, 0)
            pltpu.sync_copy(outc_v.at[pl.ds(0, CSEG)],
                            cnt_out.at[pl.ds(off, CSEG)])

            @pl.when(s == NSUB - 1)
            def _():
                pltpu.sync_copy(outc_v.at[pl.ds(CSEG, 16)],
                                cnt_out.at[pl.ds(off + CSEG, 16)])

        @pl.when(c == 0)
        def _():
            writeout(cnt0)

        @pl.when(c == 1)
        def _():
            writeout(cnt1)

    cp = pltpu.CompilerParams()
    if "needs_layout_passes" in pltpu.CompilerParams.__dataclass_fields__:
        cp = dataclasses.replace(cp, needs_layout_passes=False)
    return pl.kernel(body, out_type=out_type, mesh=mesh,
                     scratch_types=scratch, compiler_params=cp)


_sc_agg = _build_sc_agg()
_sc_count = _build_sc_count()

RB = 2000


def _build_tc_combine(first_layer):
    def body(a0, a1, c0, c1, w0, w1, b0, b1, o):
        cc0 = c0[...]
        cc1 = c1[...]
        x0 = a0[...] / jnp.maximum(cc0, 1.0)
        x1 = a1[...] / jnp.maximum(cc1, 1.0)
        t = jnp.dot(x0, w0[...], preferred_element_type=jnp.float32)
        t = t + jnp.dot(x1, w1[...], preferred_element_type=jnp.float32)
        t = t + jnp.where(cc0 > 0.0, 1.0, 0.0) * b0[...]
        t = t + jnp.where(cc1 > 0.0, 1.0, 0.0) * b1[...]
        if first_layer:
            t = jnp.where(t >= 0, t, 0.01 * t)
        o[...] = t

    return pl.pallas_call(
        body,
        grid=(N // RB,),
        in_specs=[pl.BlockSpec((RB, D), lambda i: (i, 0)),
                  pl.BlockSpec((RB, D), lambda i: (i, 0)),
                  pl.BlockSpec((RB, 1), lambda i: (i, 0)),
                  pl.BlockSpec((RB, 1), lambda i: (i, 0)),
                  pl.BlockSpec((D, D), lambda i: (0, 0)),
                  pl.BlockSpec((D, D), lambda i: (0, 0)),
                  pl.BlockSpec((1, D), lambda i: (0, 0)),
                  pl.BlockSpec((1, D), lambda i: (0, 0))],
        out_specs=pl.BlockSpec((RB, D), lambda i: (i, 0)),
        out_shape=jax.ShapeDtypeStruct((N, D), jnp.float32),
    )


_tc_combine1 = _build_tc_combine(True)
_tc_combine2 = _build_tc_combine(False)


def kernel(x, edge_index_r0, edge_index_r1, W1_0, b1_0, W1_1, b1_1,
           W2_0, b2_0, W2_1, b2_1):
    ei0 = edge_index_r0.astype(jnp.int32)
    ei1 = edge_index_r1.astype(jnp.int32)
    zpad = jnp.zeros((512,), jnp.int32)
    src_all = jnp.concatenate([ei0[0], ei1[0], zpad])
    dst_all = jnp.concatenate([ei0[1], ei1[1], zpad])
    agg0, agg1 = _sc_agg(x, src_all, dst_all)
    cnt0, cnt1 = _sc_count(dst_all)
    cnt0 = cnt0.reshape(N, 1)
    cnt1 = cnt1.reshape(N, 1)
    h = _tc_combine1(agg0, agg1, cnt0, cnt1, W1_0, W1_1,
                     b1_0.reshape(1, D), b1_1.reshape(1, D))
    agg0b, agg1b = _sc_agg(h, src_all, dst_all)
    out = _tc_combine2(agg0b, agg1b, cnt0, cnt1, W2_0, W2_1,
                       b2_0.reshape(1, D), b2_1.reshape(1, D))
    return out
